# Initial kernel scaffold; baseline (speedup 1.0000x reference)
import jax
import jax.numpy as jnp
from jax.experimental import pallas as pl


def kernel(input):
    raise NotImplementedError("write your pallas kernel here")


# speedup vs baseline: 21.7693x; 21.7693x over previous
import functools

import jax
import jax.numpy as jnp
from jax.experimental import pallas as pl
from jax.experimental.pallas import tpu as pltpu

_N_BISECT = 24
_BLOCK_ROWS = 256


def _sparsemax_block(x_ref, o_ref):
    x = x_ref[...]
    m = jnp.max(x, axis=-1, keepdims=True)
    lo = m - 1.0
    hi = m

    def bisect_step(_, carry):
        lo, hi = carry
        mid = 0.5 * (lo + hi)
        s = jnp.sum(jnp.maximum(x - mid, 0.0), axis=-1, keepdims=True)
        too_low = s >= 1.0
        lo = jnp.where(too_low, mid, lo)
        hi = jnp.where(too_low, hi, mid)
        return lo, hi

    lo, hi = jax.lax.fori_loop(0, _N_BISECT, bisect_step, (lo, hi))
    tau = 0.5 * (lo + hi)

    d = x - tau
    p = jnp.maximum(d, 0.0)
    s = jnp.sum(p, axis=-1, keepdims=True)
    c = jnp.sum((d > 0.0).astype(jnp.float32), axis=-1, keepdims=True)
    tau = tau + (s - 1.0) / jnp.maximum(c, 1.0)

    o_ref[...] = jnp.maximum(x - tau, 0.0)


def kernel(input):
    rows, cols = input.shape
    grid = (rows // _BLOCK_ROWS,)
    return pl.pallas_call(
        _sparsemax_block,
        grid=grid,
        in_specs=[pl.BlockSpec((_BLOCK_ROWS, cols), lambda i: (i, 0))],
        out_specs=pl.BlockSpec((_BLOCK_ROWS, cols), lambda i: (i, 0)),
        out_shape=jax.ShapeDtypeStruct((rows, cols), input.dtype),
        compiler_params=pltpu.CompilerParams(
            dimension_semantics=("arbitrary",),
        ),
    )(input)


# speedup vs baseline: 28.2777x; 1.2990x over previous
import functools

import jax
import jax.numpy as jnp
from jax.experimental import pallas as pl
from jax.experimental.pallas import tpu as pltpu

_N_BISECT = 16
_N_NEWTON = 2
_BLOCK_ROWS = 256


def _sparsemax_block(x_ref, o_ref):
    x = x_ref[...]
    m = jnp.max(x, axis=-1, keepdims=True)
    lo = m - 1.0
    hi = m

    def bisect_step(_, carry):
        lo, hi = carry
        mid = 0.5 * (lo + hi)
        s = jnp.sum(jnp.maximum(x - mid, 0.0), axis=-1, keepdims=True)
        too_low = s >= 1.0
        lo = jnp.where(too_low, mid, lo)
        hi = jnp.where(too_low, hi, mid)
        return lo, hi

    lo, hi = jax.lax.fori_loop(0, _N_BISECT, bisect_step, (lo, hi))
    tau = 0.5 * (lo + hi)

    def newton_step(_, tau):
        d = x - tau
        p = jnp.maximum(d, 0.0)
        s = jnp.sum(p, axis=-1, keepdims=True)
        c = jnp.sum((d > 0.0).astype(jnp.float32), axis=-1, keepdims=True)
        return tau + (s - 1.0) / jnp.maximum(c, 1.0)

    tau = jax.lax.fori_loop(0, _N_NEWTON, newton_step, tau)

    o_ref[...] = jnp.maximum(x - tau, 0.0)


def kernel(input):
    rows, cols = input.shape
    grid = (rows // _BLOCK_ROWS,)
    return pl.pallas_call(
        _sparsemax_block,
        grid=grid,
        in_specs=[pl.BlockSpec((_BLOCK_ROWS, cols), lambda i: (i, 0))],
        out_specs=pl.BlockSpec((_BLOCK_ROWS, cols), lambda i: (i, 0)),
        out_shape=jax.ShapeDtypeStruct((rows, cols), input.dtype),
        compiler_params=pltpu.CompilerParams(
            dimension_semantics=("arbitrary",),
        ),
    )(input)
